# baseline (speedup 1.0000x reference)
import jax
import jax.numpy as jnp
from jax.experimental import pallas as pl
from jax.experimental.pallas import tpu as pltpu
from jax.experimental.pallas import tpu_sc as plsc

N = 10000
NP = 10240
E = 320000
EP = 327680
D = 128
B = 128
NC = EP // 32 // B
FRAC0 = 0.25
NC0 = 40
NC1 = 2 * NC - NC0
E0 = NC0 * 16 * B
RPS = NP // 16


def _vector_mesh():
    return plsc.VectorSubcoreMesh(core_axis_name="c", subcore_axis_name="s")


def _sc_degree(dst_w, zrows, ones):
    @pl.kernel(
        out_type=jax.ShapeDtypeStruct((2, NP, D), jnp.float32),
        mesh=_vector_mesh(),
        scratch_types=[
            pltpu.VMEM((B,), jnp.int32),
            pltpu.VMEM((B, D), jnp.float32),
            pltpu.VMEM_SHARED((NP, D), jnp.float32),
        ],
    )
    def deg_kernel(dst_hbm, z_hbm, ones_hbm, out_hbm, dstb, ones_v, acc):
        cid = jax.lax.axis_index("c")
        sid = jax.lax.axis_index("s")
        wid = cid * 16 + sid
        pltpu.sync_copy(z_hbm.at[pl.ds(sid * RPS, RPS)],
                        acc.at[pl.ds(sid * RPS, RPS)])
        pltpu.sync_copy(ones_hbm, ones_v)
        plsc.subcore_barrier()

        @pl.loop(0, NC)
        def _(j):
            pltpu.sync_copy(dst_hbm.at[wid, j], dstb)
            pltpu.sync_copy(ones_v, acc.at[dstb], add=True)

        plsc.subcore_barrier()
        pltpu.sync_copy(acc.at[pl.ds(sid * RPS, RPS)],
                        out_hbm.at[cid].at[pl.ds(sid * RPS, RPS)])

    return deg_kernel(dst_w, zrows, ones)


def _sc_aggregate(h, e0, e1, zrows):
    src0, dst0 = e0
    src1, dst1 = e1

    @pl.kernel(
        out_type=jax.ShapeDtypeStruct((2, NP, D), jnp.float32),
        mesh=_vector_mesh(),
        scratch_types=[
            pltpu.VMEM((B,), jnp.int32),
            pltpu.VMEM((B,), jnp.int32),
            pltpu.VMEM((B,), jnp.int32),
            pltpu.VMEM((B,), jnp.int32),
            pltpu.VMEM((B, D), jnp.float32),
            pltpu.VMEM((B, D), jnp.float32),
            pltpu.VMEM_SHARED((NP, D), jnp.float32),
            pltpu.SemaphoreType.DMA,
            pltpu.SemaphoreType.DMA,
            pltpu.SemaphoreType.DMA,
            pltpu.SemaphoreType.DMA,
            pltpu.SemaphoreType.DMA,
            pltpu.SemaphoreType.DMA,
        ],
    )
    def agg_kernel(h_hbm, src0_hbm, dst0_hbm, src1_hbm, dst1_hbm,
                   z_hbm, out_hbm,
                   srcb0, srcb1, dstb0, dstb1, rows0, rows1, acc,
                   si0, si1, sg0, sg1, ss0, ss1):
        srcb = (srcb0, srcb1)
        dstb = (dstb0, dstb1)
        rows = (rows0, rows1)
        si = (si0, si1)
        sg = (sg0, sg1)
        ss = (ss0, ss1)
        cid = jax.lax.axis_index("c")
        sid = jax.lax.axis_index("s")
        pltpu.sync_copy(z_hbm.at[pl.ds(sid * RPS, RPS)],
                        acc.at[pl.ds(sid * RPS, RPS)])
        plsc.subcore_barrier()

        def chunk_loop(src_hbm, dst_hbm, nc):
            @pl.loop(0, nc, step=2)
            def _(j0):
                icp = []
                for u in range(2):
                    icp.append(pltpu.async_copy(src_hbm.at[sid, j0 + u],
                                                srcb[u], si[u]))
                    icp.append(pltpu.async_copy(dst_hbm.at[sid, j0 + u],
                                                dstb[u], si[u]))
                for cp in icp:
                    cp.wait()
                gcp = [pltpu.async_copy(h_hbm.at[srcb[u]], rows[u], sg[u])
                       for u in range(2)]
                scp = []
                for u in range(2):
                    gcp[u].wait()
                    scp.append(pltpu.async_copy(rows[u], acc.at[dstb[u]],
                                                ss[u], add=True))
                for cp in scp:
                    cp.wait()

        @pl.when(cid == 0)
        def _():
            chunk_loop(src0_hbm, dst0_hbm, NC0)

        @pl.when(cid == 1)
        def _():
            chunk_loop(src1_hbm, dst1_hbm, NC1)

        plsc.subcore_barrier()
        pltpu.sync_copy(acc.at[pl.ds(sid * RPS, RPS)],
                        out_hbm.at[cid].at[pl.ds(sid * RPS, RPS)])

    return agg_kernel(h, src0, dst0, src1, dst1, zrows)


_BN = 1024


def _d_col(deg_ref):
    deg = deg_ref[0, :, :1] + deg_ref[1, :, :1]
    return jax.lax.rsqrt(jnp.maximum(deg + 1.0, 1.0))


def _tc_layer1(degp, xp, W):
    def body(deg_ref, x_ref, w_ref, o_ref):
        d = _d_col(deg_ref)
        h = jnp.dot(x_ref[...], w_ref[...], preferred_element_type=jnp.float32,
                    precision=jax.lax.Precision.HIGHEST)
        o_ref[...] = h * d

    return pl.pallas_call(
        body,
        grid=(NP // _BN,),
        in_specs=[
            pl.BlockSpec((2, _BN, D), lambda i: (0, i, 0)),
            pl.BlockSpec((_BN, D), lambda i: (i, 0)),
            pl.BlockSpec((D, D), lambda i: (0, 0)),
        ],
        out_specs=pl.BlockSpec((_BN, D), lambda i: (i, 0)),
        out_shape=jax.ShapeDtypeStruct((NP, D), jnp.float32),
    )(degp, xp, W)


def _tc_mid(degp, p, hprev, bprev, W):
    def body(deg_ref, p_ref, hp_ref, b_ref, w_ref, o_ref):
        d = _d_col(deg_ref)
        s = p_ref[0] + p_ref[1] + hp_ref[...]
        xn = s * d + b_ref[...]
        h = jnp.dot(xn, w_ref[...], preferred_element_type=jnp.float32,
                    precision=jax.lax.Precision.HIGHEST)
        o_ref[...] = h * d

    return pl.pallas_call(
        body,
        grid=(NP // _BN,),
        in_specs=[
            pl.BlockSpec((2, _BN, D), lambda i: (0, i, 0)),
            pl.BlockSpec((2, _BN, D), lambda i: (0, i, 0)),
            pl.BlockSpec((_BN, D), lambda i: (i, 0)),
            pl.BlockSpec((1, D), lambda i: (0, 0)),
            pl.BlockSpec((D, D), lambda i: (0, 0)),
        ],
        out_specs=pl.BlockSpec((_BN, D), lambda i: (i, 0)),
        out_shape=jax.ShapeDtypeStruct((NP, D), jnp.float32),
    )(degp, p, hprev, bprev, W)


def _tc_out(degp, p, hprev, b):
    def body(deg_ref, p_ref, hp_ref, b_ref, o_ref):
        d = _d_col(deg_ref)
        s = p_ref[0] + p_ref[1] + hp_ref[...]
        o_ref[...] = s * d + b_ref[...]

    return pl.pallas_call(
        body,
        grid=(NP // _BN,),
        in_specs=[
            pl.BlockSpec((2, _BN, D), lambda i: (0, i, 0)),
            pl.BlockSpec((2, _BN, D), lambda i: (0, i, 0)),
            pl.BlockSpec((_BN, D), lambda i: (i, 0)),
            pl.BlockSpec((1, D), lambda i: (0, 0)),
        ],
        out_specs=pl.BlockSpec((_BN, D), lambda i: (i, 0)),
        out_shape=jax.ShapeDtypeStruct((NP, D), jnp.float32),
    )(degp, p, hprev, b)


def kernel(x, edge_index, W1, b1, W2, b2, W3, b3):
    src = edge_index[0].astype(jnp.int32)
    dst = edge_index[1].astype(jnp.int32)
    pad_e = EP - E
    src_p = jnp.concatenate([src, jnp.zeros((pad_e,), jnp.int32)])
    dst_p = jnp.concatenate([dst, jnp.full((pad_e,), N, jnp.int32)])
    e0 = (src_p[:E0].reshape(16, NC0, B), dst_p[:E0].reshape(16, NC0, B))
    e1 = (src_p[E0:].reshape(16, NC1, B), dst_p[E0:].reshape(16, NC1, B))
    dst_a = dst_p.reshape(32, NC, B)
    xp = jnp.pad(x, ((0, NP - N), (0, 0)))

    zrows = jnp.zeros((NP, D), jnp.float32)
    ones = jnp.ones((B, D), jnp.float32)

    degp = _sc_degree(dst_a, zrows, ones)

    h1 = _tc_layer1(degp, xp, W1)
    p1 = _sc_aggregate(h1, e0, e1, zrows)
    h2 = _tc_mid(degp, p1, h1, b1.reshape(1, D), W2)
    p2 = _sc_aggregate(h2, e0, e1, zrows)
    h3 = _tc_mid(degp, p2, h2, b2.reshape(1, D), W3)
    p3 = _sc_aggregate(h3, e0, e1, zrows)
    out = _tc_out(degp, p3, h3, b3.reshape(1, D))
    return out[:N]


# speedup vs baseline: 1.3039x; 1.3039x over previous
import jax
import jax.numpy as jnp
from jax.experimental import pallas as pl
from jax.experimental.pallas import tpu as pltpu
from jax.experimental.pallas import tpu_sc as plsc

N = 10000
NP = 10240
E = 320000
EP = 327680
D = 128
B = 128
NC = EP // 32 // B
FRAC0 = 0.75
NC0 = 120
NC1 = 2 * NC - NC0
E0 = NC0 * 16 * B
RPS = NP // 16


def _vector_mesh():
    return plsc.VectorSubcoreMesh(core_axis_name="c", subcore_axis_name="s")


def _sc_degree(dst_w, zrows, ones):
    @pl.kernel(
        out_type=jax.ShapeDtypeStruct((2, NP, D), jnp.float32),
        mesh=_vector_mesh(),
        scratch_types=[
            pltpu.VMEM((B,), jnp.int32),
            pltpu.VMEM((B, D), jnp.float32),
            pltpu.VMEM_SHARED((NP, D), jnp.float32),
        ],
    )
    def deg_kernel(dst_hbm, z_hbm, ones_hbm, out_hbm, dstb, ones_v, acc):
        cid = jax.lax.axis_index("c")
        sid = jax.lax.axis_index("s")
        wid = cid * 16 + sid
        pltpu.sync_copy(z_hbm.at[pl.ds(sid * RPS, RPS)],
                        acc.at[pl.ds(sid * RPS, RPS)])
        pltpu.sync_copy(ones_hbm, ones_v)
        plsc.subcore_barrier()

        @pl.loop(0, NC)
        def _(j):
            pltpu.sync_copy(dst_hbm.at[wid, j], dstb)
            pltpu.sync_copy(ones_v, acc.at[dstb], add=True)

        plsc.subcore_barrier()
        pltpu.sync_copy(acc.at[pl.ds(sid * RPS, RPS)],
                        out_hbm.at[cid].at[pl.ds(sid * RPS, RPS)])

    return deg_kernel(dst_w, zrows, ones)


def _sc_aggregate(h, e0, e1, zrows):
    src0, dst0 = e0
    src1, dst1 = e1

    @pl.kernel(
        out_type=jax.ShapeDtypeStruct((2, NP, D), jnp.float32),
        mesh=_vector_mesh(),
        scratch_types=[
            pltpu.VMEM((B,), jnp.int32),
            pltpu.VMEM((B,), jnp.int32),
            pltpu.VMEM((B,), jnp.int32),
            pltpu.VMEM((B,), jnp.int32),
            pltpu.VMEM((B, D), jnp.float32),
            pltpu.VMEM((B, D), jnp.float32),
            pltpu.VMEM_SHARED((NP, D), jnp.float32),
            pltpu.SemaphoreType.DMA,
            pltpu.SemaphoreType.DMA,
            pltpu.SemaphoreType.DMA,
            pltpu.SemaphoreType.DMA,
            pltpu.SemaphoreType.DMA,
            pltpu.SemaphoreType.DMA,
        ],
    )
    def agg_kernel(h_hbm, src0_hbm, dst0_hbm, src1_hbm, dst1_hbm,
                   z_hbm, out_hbm,
                   srcb0, srcb1, dstb0, dstb1, rows0, rows1, acc,
                   si0, si1, sg0, sg1, ss0, ss1):
        srcb = (srcb0, srcb1)
        dstb = (dstb0, dstb1)
        rows = (rows0, rows1)
        si = (si0, si1)
        sg = (sg0, sg1)
        ss = (ss0, ss1)
        cid = jax.lax.axis_index("c")
        sid = jax.lax.axis_index("s")
        pltpu.sync_copy(z_hbm.at[pl.ds(sid * RPS, RPS)],
                        acc.at[pl.ds(sid * RPS, RPS)])
        plsc.subcore_barrier()

        def chunk_loop(src_hbm, dst_hbm, nc):
            @pl.loop(0, nc, step=2)
            def _(j0):
                icp = []
                for u in range(2):
                    icp.append(pltpu.async_copy(src_hbm.at[sid, j0 + u],
                                                srcb[u], si[u]))
                    icp.append(pltpu.async_copy(dst_hbm.at[sid, j0 + u],
                                                dstb[u], si[u]))
                for cp in icp:
                    cp.wait()
                gcp = [pltpu.async_copy(h_hbm.at[srcb[u]], rows[u], sg[u])
                       for u in range(2)]
                scp = []
                for u in range(2):
                    gcp[u].wait()
                    scp.append(pltpu.async_copy(rows[u], acc.at[dstb[u]],
                                                ss[u], add=True))
                for cp in scp:
                    cp.wait()

        @pl.when(cid == 0)
        def _():
            chunk_loop(src0_hbm, dst0_hbm, NC0)

        @pl.when(cid == 1)
        def _():
            chunk_loop(src1_hbm, dst1_hbm, NC1)

        plsc.subcore_barrier()
        pltpu.sync_copy(acc.at[pl.ds(sid * RPS, RPS)],
                        out_hbm.at[cid].at[pl.ds(sid * RPS, RPS)])

    return agg_kernel(h, src0, dst0, src1, dst1, zrows)


_BN = 1024


def _d_col(deg_ref):
    deg = deg_ref[0, :, :1] + deg_ref[1, :, :1]
    return jax.lax.rsqrt(jnp.maximum(deg + 1.0, 1.0))


def _tc_layer1(degp, xp, W):
    def body(deg_ref, x_ref, w_ref, o_ref):
        d = _d_col(deg_ref)
        h = jnp.dot(x_ref[...], w_ref[...], preferred_element_type=jnp.float32,
                    precision=jax.lax.Precision.HIGHEST)
        o_ref[...] = h * d

    return pl.pallas_call(
        body,
        grid=(NP // _BN,),
        in_specs=[
            pl.BlockSpec((2, _BN, D), lambda i: (0, i, 0)),
            pl.BlockSpec((_BN, D), lambda i: (i, 0)),
            pl.BlockSpec((D, D), lambda i: (0, 0)),
        ],
        out_specs=pl.BlockSpec((_BN, D), lambda i: (i, 0)),
        out_shape=jax.ShapeDtypeStruct((NP, D), jnp.float32),
    )(degp, xp, W)


def _tc_mid(degp, p, hprev, bprev, W):
    def body(deg_ref, p_ref, hp_ref, b_ref, w_ref, o_ref):
        d = _d_col(deg_ref)
        s = p_ref[0] + p_ref[1] + hp_ref[...]
        xn = s * d + b_ref[...]
        h = jnp.dot(xn, w_ref[...], preferred_element_type=jnp.float32,
                    precision=jax.lax.Precision.HIGHEST)
        o_ref[...] = h * d

    return pl.pallas_call(
        body,
        grid=(NP // _BN,),
        in_specs=[
            pl.BlockSpec((2, _BN, D), lambda i: (0, i, 0)),
            pl.BlockSpec((2, _BN, D), lambda i: (0, i, 0)),
            pl.BlockSpec((_BN, D), lambda i: (i, 0)),
            pl.BlockSpec((1, D), lambda i: (0, 0)),
            pl.BlockSpec((D, D), lambda i: (0, 0)),
        ],
        out_specs=pl.BlockSpec((_BN, D), lambda i: (i, 0)),
        out_shape=jax.ShapeDtypeStruct((NP, D), jnp.float32),
    )(degp, p, hprev, bprev, W)


def _tc_out(degp, p, hprev, b):
    def body(deg_ref, p_ref, hp_ref, b_ref, o_ref):
        d = _d_col(deg_ref)
        s = p_ref[0] + p_ref[1] + hp_ref[...]
        o_ref[...] = s * d + b_ref[...]

    return pl.pallas_call(
        body,
        grid=(NP // _BN,),
        in_specs=[
            pl.BlockSpec((2, _BN, D), lambda i: (0, i, 0)),
            pl.BlockSpec((2, _BN, D), lambda i: (0, i, 0)),
            pl.BlockSpec((_BN, D), lambda i: (i, 0)),
            pl.BlockSpec((1, D), lambda i: (0, 0)),
        ],
        out_specs=pl.BlockSpec((_BN, D), lambda i: (i, 0)),
        out_shape=jax.ShapeDtypeStruct((NP, D), jnp.float32),
    )(degp, p, hprev, b)


def kernel(x, edge_index, W1, b1, W2, b2, W3, b3):
    src = edge_index[0].astype(jnp.int32)
    dst = edge_index[1].astype(jnp.int32)
    pad_e = EP - E
    src_p = jnp.concatenate([src, jnp.zeros((pad_e,), jnp.int32)])
    dst_p = jnp.concatenate([dst, jnp.full((pad_e,), N, jnp.int32)])
    e0 = (src_p[:E0].reshape(16, NC0, B), dst_p[:E0].reshape(16, NC0, B))
    e1 = (src_p[E0:].reshape(16, NC1, B), dst_p[E0:].reshape(16, NC1, B))
    dst_a = dst_p.reshape(32, NC, B)
    xp = jnp.pad(x, ((0, NP - N), (0, 0)))

    zrows = jnp.zeros((NP, D), jnp.float32)
    ones = jnp.ones((B, D), jnp.float32)

    degp = _sc_degree(dst_a, zrows, ones)

    h1 = _tc_layer1(degp, xp, W1)
    p1 = _sc_aggregate(h1, e0, e1, zrows)
    h2 = _tc_mid(degp, p1, h1, b1.reshape(1, D), W2)
    p2 = _sc_aggregate(h2, e0, e1, zrows)
    h3 = _tc_mid(degp, p2, h2, b2.reshape(1, D), W3)
    p3 = _sc_aggregate(h3, e0, e1, zrows)
    out = _tc_out(degp, p3, h3, b3.reshape(1, D))
    return out[:N]


# speedup vs baseline: 1.3048x; 1.0006x over previous
import jax
import jax.numpy as jnp
from jax.experimental import pallas as pl
from jax.experimental.pallas import tpu as pltpu
from jax.experimental.pallas import tpu_sc as plsc

N = 10000
NP = 10240
E = 320000
EP = 327680
D = 128
B = 128
NC = EP // 32 // B
FRAC0 = 0.75
NC0 = 120
NC1 = 2 * NC - NC0
E0 = NC0 * 16 * B
RPS = NP // 16


def _vector_mesh():
    return plsc.VectorSubcoreMesh(core_axis_name="c", subcore_axis_name="s")


def _sc_degree(dst_w, zrows, ones):
    @pl.kernel(
        out_type=jax.ShapeDtypeStruct((2, NP, D), jnp.float32),
        mesh=_vector_mesh(),
        scratch_types=[
            pltpu.VMEM((B,), jnp.int32),
            pltpu.VMEM((B, D), jnp.float32),
            pltpu.VMEM_SHARED((NP, D), jnp.float32),
        ],
    )
    def deg_kernel(dst_hbm, z_hbm, ones_hbm, out_hbm, dstb, ones_v, acc):
        cid = jax.lax.axis_index("c")
        sid = jax.lax.axis_index("s")
        wid = cid * 16 + sid
        pltpu.sync_copy(z_hbm.at[pl.ds(sid * RPS, RPS)],
                        acc.at[pl.ds(sid * RPS, RPS)])
        pltpu.sync_copy(ones_hbm, ones_v)
        plsc.subcore_barrier()

        @pl.loop(0, NC)
        def _(j):
            pltpu.sync_copy(dst_hbm.at[wid, j], dstb)
            pltpu.sync_copy(ones_v, acc.at[dstb], add=True)

        plsc.subcore_barrier()
        pltpu.sync_copy(acc.at[pl.ds(sid * RPS, RPS)],
                        out_hbm.at[cid].at[pl.ds(sid * RPS, RPS)])

    return deg_kernel(dst_w, zrows, ones)


def _sc_aggregate(h, e0, e1, zrows):
    src0, dst0 = e0
    src1, dst1 = (e1 if NC1 else e0)

    @pl.kernel(
        out_type=jax.ShapeDtypeStruct((2, NP, D), jnp.float32),
        mesh=_vector_mesh(),
        scratch_types=[
            pltpu.VMEM((B,), jnp.int32),
            pltpu.VMEM((B,), jnp.int32),
            pltpu.VMEM((B,), jnp.int32),
            pltpu.VMEM((B,), jnp.int32),
            pltpu.VMEM((B, D), jnp.float32),
            pltpu.VMEM((B, D), jnp.float32),
            pltpu.VMEM_SHARED((NP, D), jnp.float32),
            pltpu.SemaphoreType.DMA,
            pltpu.SemaphoreType.DMA,
            pltpu.SemaphoreType.DMA,
            pltpu.SemaphoreType.DMA,
            pltpu.SemaphoreType.DMA,
            pltpu.SemaphoreType.DMA,
        ],
    )
    def agg_kernel(h_hbm, src0_hbm, dst0_hbm, src1_hbm, dst1_hbm,
                   z_hbm, out_hbm,
                   srcb0, srcb1, dstb0, dstb1, rows0, rows1, acc,
                   si0, si1, sg0, sg1, ss0, ss1):
        srcb = (srcb0, srcb1)
        dstb = (dstb0, dstb1)
        rows = (rows0, rows1)
        si = (si0, si1)
        sg = (sg0, sg1)
        ss = (ss0, ss1)
        cid = jax.lax.axis_index("c")
        sid = jax.lax.axis_index("s")
        pltpu.sync_copy(z_hbm.at[pl.ds(sid * RPS, RPS)],
                        acc.at[pl.ds(sid * RPS, RPS)])
        plsc.subcore_barrier()

        def chunk_loop(src_hbm, dst_hbm, nc):
            @pl.loop(0, nc, step=2)
            def _(j0):
                icp = []
                for u in range(2):
                    icp.append(pltpu.async_copy(src_hbm.at[sid, j0 + u],
                                                srcb[u], si[u]))
                    icp.append(pltpu.async_copy(dst_hbm.at[sid, j0 + u],
                                                dstb[u], si[u]))
                for cp in icp:
                    cp.wait()
                gcp = [pltpu.async_copy(h_hbm.at[srcb[u]], rows[u], sg[u])
                       for u in range(2)]
                scp = []
                for u in range(2):
                    gcp[u].wait()
                    scp.append(pltpu.async_copy(rows[u], acc.at[dstb[u]],
                                                ss[u], add=True))
                for cp in scp:
                    cp.wait()

        @pl.when(cid == 0)
        def _():
            chunk_loop(src0_hbm, dst0_hbm, NC0)

        if NC1:
            @pl.when(cid == 1)
            def _():
                chunk_loop(src1_hbm, dst1_hbm, NC1)

        plsc.subcore_barrier()
        pltpu.sync_copy(acc.at[pl.ds(sid * RPS, RPS)],
                        out_hbm.at[cid].at[pl.ds(sid * RPS, RPS)])

    return agg_kernel(h, src0, dst0, src1, dst1, zrows)


_BN = 1024


def _d_col(deg_ref):
    deg = deg_ref[0, :, :1] + deg_ref[1, :, :1]
    return jax.lax.rsqrt(jnp.maximum(deg + 1.0, 1.0))


def _tc_layer1(degp, xp, W):
    def body(deg_ref, x_ref, w_ref, o_ref):
        d = _d_col(deg_ref)
        h = jnp.dot(x_ref[...], w_ref[...], preferred_element_type=jnp.float32,
                    precision=jax.lax.Precision.HIGHEST)
        o_ref[...] = h * d

    return pl.pallas_call(
        body,
        grid=(NP // _BN,),
        in_specs=[
            pl.BlockSpec((2, _BN, D), lambda i: (0, i, 0)),
            pl.BlockSpec((_BN, D), lambda i: (i, 0)),
            pl.BlockSpec((D, D), lambda i: (0, 0)),
        ],
        out_specs=pl.BlockSpec((_BN, D), lambda i: (i, 0)),
        out_shape=jax.ShapeDtypeStruct((NP, D), jnp.float32),
    )(degp, xp, W)


def _tc_mid(degp, p, hprev, bprev, W):
    def body(deg_ref, p_ref, hp_ref, b_ref, w_ref, o_ref):
        d = _d_col(deg_ref)
        s = p_ref[0] + p_ref[1] + hp_ref[...]
        xn = s * d + b_ref[...]
        h = jnp.dot(xn, w_ref[...], preferred_element_type=jnp.float32,
                    precision=jax.lax.Precision.HIGHEST)
        o_ref[...] = h * d

    return pl.pallas_call(
        body,
        grid=(NP // _BN,),
        in_specs=[
            pl.BlockSpec((2, _BN, D), lambda i: (0, i, 0)),
            pl.BlockSpec((2, _BN, D), lambda i: (0, i, 0)),
            pl.BlockSpec((_BN, D), lambda i: (i, 0)),
            pl.BlockSpec((1, D), lambda i: (0, 0)),
            pl.BlockSpec((D, D), lambda i: (0, 0)),
        ],
        out_specs=pl.BlockSpec((_BN, D), lambda i: (i, 0)),
        out_shape=jax.ShapeDtypeStruct((NP, D), jnp.float32),
    )(degp, p, hprev, bprev, W)


def _tc_out(degp, p, hprev, b):
    def body(deg_ref, p_ref, hp_ref, b_ref, o_ref):
        d = _d_col(deg_ref)
        s = p_ref[0] + p_ref[1] + hp_ref[...]
        o_ref[...] = s * d + b_ref[...]

    return pl.pallas_call(
        body,
        grid=(NP // _BN,),
        in_specs=[
            pl.BlockSpec((2, _BN, D), lambda i: (0, i, 0)),
            pl.BlockSpec((2, _BN, D), lambda i: (0, i, 0)),
            pl.BlockSpec((_BN, D), lambda i: (i, 0)),
            pl.BlockSpec((1, D), lambda i: (0, 0)),
        ],
        out_specs=pl.BlockSpec((_BN, D), lambda i: (i, 0)),
        out_shape=jax.ShapeDtypeStruct((NP, D), jnp.float32),
    )(degp, p, hprev, b)


def kernel(x, edge_index, W1, b1, W2, b2, W3, b3):
    src = edge_index[0].astype(jnp.int32)
    dst = edge_index[1].astype(jnp.int32)
    pad_e = EP - E
    src_p = jnp.concatenate([src, jnp.zeros((pad_e,), jnp.int32)])
    dst_p = jnp.concatenate([dst, jnp.full((pad_e,), N, jnp.int32)])
    e0 = (src_p[:E0].reshape(16, NC0, B), dst_p[:E0].reshape(16, NC0, B))
    e1 = ((src_p[E0:].reshape(16, NC1, B), dst_p[E0:].reshape(16, NC1, B))
          if NC1 else None)
    dst_a = dst_p.reshape(32, NC, B)
    xp = jnp.pad(x, ((0, NP - N), (0, 0)))

    zrows = jnp.zeros((NP, D), jnp.float32)
    ones = jnp.ones((B, D), jnp.float32)

    degp = _sc_degree(dst_a, zrows, ones)

    h1 = _tc_layer1(degp, xp, W1)
    p1 = _sc_aggregate(h1, e0, e1, zrows)
    h2 = _tc_mid(degp, p1, h1, b1.reshape(1, D), W2)
    p2 = _sc_aggregate(h2, e0, e1, zrows)
    h3 = _tc_mid(degp, p2, h2, b2.reshape(1, D), W3)
    p3 = _sc_aggregate(h3, e0, e1, zrows)
    out = _tc_out(degp, p3, h3, b3.reshape(1, D))
    return out[:N]
